# Initial kernel scaffold; baseline (speedup 1.0000x reference)
import jax
import jax.numpy as jnp
from jax.experimental import pallas as pl


def kernel(node_features, edge_index, batch, W0, as0, ad0, b0, lnw0, lnb0, W1, as1, ad1, b1, lnw1, lnb1, W2, as2, ad2, b2, lnw2, lnb2, pW, pb, plnw, plnb, poolW, poolb):
    raise NotImplementedError("write your pallas kernel here")


# speedup vs baseline: 19.4373x; 19.4373x over previous
import jax
import jax.numpy as jnp
from jax import lax
from jax.experimental import pallas as pl
from jax.experimental.pallas import tpu as pltpu
from jax.experimental.pallas import tpu_sc as plsc

N = 10000
E = 160000
DIN = 128
H = 4
C = 64
HC = H * C
DOUT = 128
G = 16

NC = 2
NS = 16
NW = NC * NS
L = 128

NP = N + 8
EP = 163840
ER = EP // L

RW1 = ER // NW
BR1 = 8
NB1 = RW1 // BR1
RW2 = ER // NS
BR2 = 2
NB2 = RW2 // BR2
B1 = BR1 * L
B2 = BR2 * L

R0 = 624
REM = N - R0 * NS
REMZ = NP - R0 * NS


def _mesh():
    return plsc.VectorSubcoreMesh(
        core_axis_name="c", subcore_axis_name="s", num_cores=NC, num_subcores=NS)


_SC_PARAMS = pltpu.CompilerParams(
    use_tc_tiling_on_sc=False, needs_layout_passes=False)

_TC_PARAMS = pltpu.CompilerParams(vmem_limit_bytes=100 * 1024 * 1024)


def _ln(x, w, b, eps=1e-5):
    mu = jnp.mean(x, axis=-1, keepdims=True)
    var = jnp.mean((x - mu) ** 2, axis=-1, keepdims=True)
    return (x - mu) / jnp.sqrt(var + eps) * w + b


def _elu(x):
    return jnp.where(x > 0, x, jnp.exp(jnp.minimum(x, 0.0)) - 1.0)


def _emit_tables(xw, ms, md, xwt_ref, astab_ref, adtab_ref):
    zpad8 = jnp.zeros((8, 128), jnp.float32)
    xwt_ref[0, pl.ds(0, N)] = xw[:, :128]
    xwt_ref[0, pl.ds(N, 8)] = zpad8
    xwt_ref[1, pl.ds(0, N)] = xw[:, 128:]
    xwt_ref[1, pl.ds(N, 8)] = zpad8
    astab_ref[pl.ds(0, N), :] = jnp.dot(xw, ms, preferred_element_type=jnp.float32)
    astab_ref[pl.ds(N, 8), :] = jnp.zeros((8, 16), jnp.float32)
    adtab_ref[pl.ds(0, N), :] = jnp.dot(xw, md, preferred_element_type=jnp.float32)
    adtab_ref[pl.ds(N, 8), :] = jnp.zeros((8, 16), jnp.float32)


def _pre_body(x_ref, w_ref, ms_ref, md_ref, xwt_ref, astab_ref, adtab_ref):
    xw = jnp.dot(x_ref[...], w_ref[...], preferred_element_type=jnp.float32)
    _emit_tables(xw, ms_ref[...], md_ref[...], xwt_ref, astab_ref, adtab_ref)


def _gat_post(out_ref, dpart_ref, b, lnw, lnb, concat):
    dn = dpart_ref[0] + dpart_ref[1]
    halves = (out_ref[0], out_ref[1])
    parts = []
    for h in range(H):
        blk = halves[h // 2][:, (h % 2) * C:(h % 2 + 1) * C]
        parts.append(blk * (1.0 / (dn[:, h:h + 1] + 1e-16)))
    if concat:
        out = jnp.concatenate(parts, axis=1)
    else:
        out = (parts[0] + parts[1] + parts[2] + parts[3]) * 0.25
    return _ln(_elu(out + b), lnw, lnb)


def _mid_body(out_ref, dpart_ref, b_ref, lnw_ref, lnb_ref, w_ref, ms_ref,
              md_ref, xwt_ref, astab_ref, adtab_ref):
    x = _gat_post(out_ref, dpart_ref, b_ref[...], lnw_ref[...], lnb_ref[...],
                  concat=True)
    xw = jnp.dot(x, w_ref[...], preferred_element_type=jnp.float32)
    _emit_tables(xw, ms_ref[...], md_ref[...], xwt_ref, astab_ref, adtab_ref)


def _final_body(out_ref, dpart_ref, b_ref, lnw_ref, lnb_ref, pw_ref, pb_ref,
                plnw_ref, plnb_ref, poolw_ref, poolb_ref, batch_ref,
                graph_ref, nodeout_ref):
    x = _gat_post(out_ref, dpart_ref, b_ref[...], lnw_ref[...], lnb_ref[...],
                  concat=False)
    z = jnp.dot(x, pw_ref[...], preferred_element_type=jnp.float32) + pb_ref[...]
    z = _ln(z, plnw_ref[...], plnb_ref[...])
    node_out = 0.5 * z * (1.0 + lax.erf(z * (2.0 ** -0.5)))
    nodeout_ref[...] = node_out
    s = jnp.dot(node_out, poolw_ref[...],
                preferred_element_type=jnp.float32) + poolb_ref[...]
    es = jnp.exp(s - jnp.max(s))
    pw = es / jnp.sum(es)
    oh = (lax.broadcasted_iota(jnp.int32, (G, N), 0)
          == batch_ref[...]).astype(jnp.float32)
    num = jnp.dot(oh, node_out * pw, preferred_element_type=jnp.float32)
    den = jnp.dot(oh, pw, preferred_element_type=jnp.float32)
    graph_ref[...] = num / jnp.maximum(den, 1e-8)


def _make_pre(interpret=False):
    return pl.pallas_call(
        _pre_body,
        out_shape=(jax.ShapeDtypeStruct((NC, NP, 128), jnp.float32),
                   jax.ShapeDtypeStruct((NP, 16), jnp.float32),
                   jax.ShapeDtypeStruct((NP, 16), jnp.float32)),
        compiler_params=_TC_PARAMS,
        interpret=interpret,
    )


def _make_mid(interpret=False):
    return pl.pallas_call(
        _mid_body,
        out_shape=(jax.ShapeDtypeStruct((NC, NP, 128), jnp.float32),
                   jax.ShapeDtypeStruct((NP, 16), jnp.float32),
                   jax.ShapeDtypeStruct((NP, 16), jnp.float32)),
        compiler_params=_TC_PARAMS,
        interpret=interpret,
    )


def _make_final(interpret=False):
    return pl.pallas_call(
        _final_body,
        out_shape=(jax.ShapeDtypeStruct((G, DOUT), jnp.float32),
                   jax.ShapeDtypeStruct((N, DOUT), jnp.float32)),
        compiler_params=_TC_PARAMS,
        interpret=interpret,
    )


def _sc1_body(astab, adtab, src_h, dst_h, ea_h, dpart_h,
              srcv, dstv, asg, adg, eav, dsh, sem1, sem2):
    c = lax.axis_index("c")
    s = lax.axis_index("s")
    w = c * NS + s
    zero16 = jnp.zeros((16,), jnp.float32)

    def zrow(i, _):
        eav[i, :] = zero16
        return 0
    lax.fori_loop(0, R0, zrow, 0)
    pltpu.sync_copy(eav.at[pl.ds(0, R0)], dsh.at[pl.ds(s * R0, R0)])

    @pl.when(s == NS - 1)
    def _():
        pltpu.sync_copy(eav.at[pl.ds(0, REMZ)], dsh.at[pl.ds(NS * R0, REMZ)])
    plsc.subcore_barrier()

    def batch(i, _):
        row0 = w * RW1 + i * BR1
        ebase = row0 * L
        pltpu.sync_copy(src_h.at[pl.ds(row0, BR1)], srcv)
        pltpu.sync_copy(dst_h.at[pl.ds(row0, BR1)], dstv)
        for j in range(BR1):
            cp1 = pltpu.async_copy(
                astab.at[srcv.at[j]], asg.at[pl.ds(j * L, L)], sem1)
            cp2 = pltpu.async_copy(
                adtab.at[dstv.at[j]], adg.at[pl.ds(j * L, L)], sem2)
            cp1.wait()
            cp2.wait()

        def edge(e, _):
            a = asg[e, :] + adg[e, :]
            a = jnp.maximum(a, 0.2 * a)
            eav[e, :] = jnp.exp(a)
            return 0
        lax.fori_loop(0, B1, edge, 0)
        pltpu.sync_copy(eav, ea_h.at[pl.ds(ebase, B1)])
        for j in range(BR1):
            pltpu.sync_copy(eav.at[pl.ds(j * L, L)], dsh.at[dstv.at[j]],
                            add=True)
        return 0
    lax.fori_loop(0, NB1, batch, 0)
    plsc.subcore_barrier()
    pltpu.sync_copy(dsh.at[pl.ds(s * R0, R0)],
                    dpart_h.at[pl.ds(c * N + s * R0, R0)])

    @pl.when(s == NS - 1)
    def _():
        pltpu.sync_copy(dsh.at[pl.ds(NS * R0, REM)],
                        dpart_h.at[pl.ds(c * N + NS * R0, REM)])


def _make_sc1(interpret=False):
    return pl.kernel(
        _sc1_body,
        out_type=(jax.ShapeDtypeStruct((EP, 16), jnp.float32),
                  jax.ShapeDtypeStruct((NC * N, 16), jnp.float32)),
        mesh=_mesh(),
        scratch_types=[
            pltpu.VMEM((BR1, L), jnp.int32),
            pltpu.VMEM((BR1, L), jnp.int32),
            pltpu.VMEM((B1, 16), jnp.float32),
            pltpu.VMEM((B1, 16), jnp.float32),
            pltpu.VMEM((B1, 16), jnp.float32),
            pltpu.VMEM_SHARED((NP, 16), jnp.float32),
            pltpu.SemaphoreType.DMA,
            pltpu.SemaphoreType.DMA,
        ],
        compiler_params=_SC_PARAMS,
        interpret=interpret,
    )


def _sc2_body(xwt_h, src_h, dst_h, ea_h, out_h,
              srcv, sadj, dstv, eag, xwg, osh, sem1):
    c = lax.axis_index("c")
    s = lax.axis_index("s")
    zero16 = jnp.zeros((16,), jnp.float32)
    cnp = c * NP

    def zrow(i, _):
        for j in range(8):
            xwg[i, pl.ds(16 * j, 16)] = zero16
        return 0
    lax.fori_loop(0, B2, zrow, 0)
    for (off, sz) in ((0, B2), (B2, B2), (2 * B2, R0 - 2 * B2)):
        pltpu.sync_copy(xwg.at[pl.ds(0, sz)], osh.at[pl.ds(s * R0 + off, sz)])

    @pl.when(s == NS - 1)
    def _():
        pltpu.sync_copy(xwg.at[pl.ds(0, REMZ)], osh.at[pl.ds(NS * R0, REMZ)])
    plsc.subcore_barrier()

    lane0 = 2 * c

    def batch(i, _):
        row0 = s * RW2 + i * BR2
        ebase = row0 * L
        pltpu.sync_copy(src_h.at[pl.ds(row0, BR2)], srcv)
        pltpu.sync_copy(dst_h.at[pl.ds(row0, BR2)], dstv)
        pltpu.sync_copy(ea_h.at[pl.ds(ebase, B2)], eag)
        for j in range(BR2):
            for k in range(L // 16):
                sadj[j, pl.ds(16 * k, 16)] = srcv[j, pl.ds(16 * k, 16)] + cnp
        for j in range(BR2):
            pltpu.async_copy(xwt_h.at[sadj.at[j]],
                             xwg.at[pl.ds(j * L, L)], sem1).wait()

        def edge(e, _):
            rowi = jnp.full((16,), e, jnp.int32)
            s0 = plsc.load_gather(eag, [rowi, jnp.full((16,), lane0, jnp.int32)])
            s1 = plsc.load_gather(eag, [rowi, jnp.full((16,), lane0 + 1, jnp.int32)])
            for j in range(8):
                sp = s0 if j < 4 else s1
                xwg[e, pl.ds(16 * j, 16)] = xwg[e, pl.ds(16 * j, 16)] * sp
            return 0
        lax.fori_loop(0, B2, edge, 0)
        for j in range(BR2):
            pltpu.sync_copy(xwg.at[pl.ds(j * L, L)], osh.at[dstv.at[j]],
                            add=True)
        return 0
    lax.fori_loop(0, NB2, batch, 0)
    plsc.subcore_barrier()
    pltpu.sync_copy(osh.at[pl.ds(s * R0, R0)],
                    out_h.at[pl.ds(c * N + s * R0, R0)])

    @pl.when(s == NS - 1)
    def _():
        pltpu.sync_copy(osh.at[pl.ds(NS * R0, REM)],
                        out_h.at[pl.ds(c * N + NS * R0, REM)])


def _make_sc2(interpret=False):
    return pl.kernel(
        _sc2_body,
        out_type=jax.ShapeDtypeStruct((NC * N, 128), jnp.float32),
        mesh=_mesh(),
        scratch_types=[
            pltpu.VMEM((BR2, L), jnp.int32),
            pltpu.VMEM((BR2, L), jnp.int32),
            pltpu.VMEM((BR2, L), jnp.int32),
            pltpu.VMEM((B2, 16), jnp.float32),
            pltpu.VMEM((B2, 128), jnp.float32),
            pltpu.VMEM_SHARED((NP, 128), jnp.float32),
            pltpu.SemaphoreType.DMA,
        ],
        compiler_params=_SC_PARAMS,
        interpret=interpret,
    )


def _attn_mats(a_s, a_d):
    k = jnp.arange(HC)
    ms = jnp.zeros((HC, 16), jnp.float32).at[k, k // C].set(a_s.reshape(-1))
    md = jnp.zeros((HC, 16), jnp.float32).at[k, k // C].set(a_d.reshape(-1))
    return ms, md


def _run(inputs, interpret=False):
    (node_features, edge_index, batch,
     W0, as0, ad0, b0, lnw0, lnb0,
     W1, as1, ad1, b1, lnw1, lnb1,
     W2, as2, ad2, b2, lnw2, lnb2,
     pW, pb, plnw, plnb, poolW, poolb) = inputs
    pad = jnp.full((EP - E,), N, jnp.int32)
    src2 = jnp.concatenate([edge_index[0], pad]).reshape(ER, L)
    dst2 = jnp.concatenate([edge_index[1], pad]).reshape(ER, L)
    batch_r = batch.reshape(1, N)

    def sc1(astab, adtab, s2, d2):
        ea, dp = _make_sc1(interpret)(astab, adtab, s2, d2)
        return ea, dp.reshape(NC, N, 16)

    def sc2(xwtf, s2, d2, ea):
        return _make_sc2(interpret)(xwtf, s2, d2, ea).reshape(NC, N, 128)

    ms0, md0 = _attn_mats(as0, ad0)
    ms1, md1 = _attn_mats(as1, ad1)
    ms2, md2 = _attn_mats(as2, ad2)

    pre = _make_pre(interpret)
    mid = _make_mid(interpret)
    fin = _make_final(interpret)

    xwt, astab, adtab = pre(node_features, W0, ms0, md0)

    ea, dpart = sc1(astab, adtab, src2, dst2)
    out = sc2(xwt.reshape(NC * NP, 128), src2, dst2, ea)
    xwt, astab, adtab = mid(
        out, dpart, b0.reshape(1, HC), lnw0.reshape(1, HC), lnb0.reshape(1, HC),
        W1, ms1, md1)

    ea, dpart = sc1(astab, adtab, src2, dst2)
    out = sc2(xwt.reshape(NC * NP, 128), src2, dst2, ea)
    xwt, astab, adtab = mid(
        out, dpart, b1.reshape(1, HC), lnw1.reshape(1, HC), lnb1.reshape(1, HC),
        W2, ms2, md2)

    ea, dpart = sc1(astab, adtab, src2, dst2)
    out = sc2(xwt.reshape(NC * NP, 128), src2, dst2, ea)
    graph, node_out = fin(
        out, dpart, b2.reshape(1, C), lnw2.reshape(1, C), lnb2.reshape(1, C),
        pW, pb.reshape(1, DOUT), plnw.reshape(1, DOUT), plnb.reshape(1, DOUT),
        poolW, poolb.reshape(1, 1), batch_r)
    return (graph, node_out)


def kernel(node_features, edge_index, batch,
           W0, as0, ad0, b0, lnw0, lnb0,
           W1, as1, ad1, b1, lnw1, lnb1,
           W2, as2, ad2, b2, lnw2, lnb2,
           pW, pb, plnw, plnb, poolW, poolb):
    return _run((node_features, edge_index, batch,
                 W0, as0, ad0, b0, lnw0, lnb0,
                 W1, as1, ad1, b1, lnw1, lnb1,
                 W2, as2, ad2, b2, lnw2, lnb2,
                 pW, pb, plnw, plnb, poolW, poolb))
